# baseline (speedup 1.0000x reference)
import functools

import jax
import jax.numpy as jnp
from jax import lax
from jax.experimental import pallas as pl
from jax.experimental.pallas import tpu as pltpu
from jax.experimental.pallas import tpu_sc as plsc

PB = 512
GROUP = 8


def _build_sc_kernel(sq, sv, K, D):
    mp = (K - 1) // 2
    info = plsc.get_sparse_core_info()
    NC, NS = info.num_cores, info.num_subcores
    NW = NC * NS
    rows_per_w = sq // NW
    chunks_per_row = sv // PB
    total = rows_per_w * chunks_per_row
    n_groups = total // GROUP
    clen = 2 * PB + K

    mesh = plsc.VectorSubcoreMesh(core_axis_name="c", subcore_axis_name="s")

    @functools.partial(
        pl.kernel, mesh=mesh,
        out_type=jax.ShapeDtypeStruct((sq, sv, D), jnp.float32),
        scratch_types=[
            pltpu.VMEM((clen, D), jnp.float32),
            pltpu.SemaphoreType.DMA,
        ],
        compiler_params=pltpu.CompilerParams(use_tc_tiling_on_sc=False),
    )
    def k(emb_hbm, out_hbm, c_small, sem):
        wid = lax.axis_index("s") * NC + lax.axis_index("c")
        base = wid * rows_per_w

        pltpu.sync_copy(emb_hbm, c_small.at[pl.ds(PB, K), :])

        def fill_body(r, _):
            for v in range(D // 16):
                sl = pl.ds(16 * v, 16)
                c_small[r, sl] = c_small[PB, sl]
                c_small[PB + K + r, sl] = c_small[PB + K - 1, sl]
            return 0

        lax.fori_loop(0, PB, fill_body, 0)

        def group_body(g, _):
            copies = []
            for b in range(GROUP):
                t = g * GROUP + b
                r = t // chunks_per_row
                ch = t % chunks_per_row
                i = base + r
                j0 = ch * PB
                off = jnp.clip(j0 - i + mp + PB, 0, PB + K)
                copies.append(pltpu.async_copy(
                    c_small.at[pl.ds(off, PB), :],
                    out_hbm.at[i, pl.ds(j0, PB), :],
                    sem))
            for c in copies:
                c.wait()
            return 0

        lax.fori_loop(0, n_groups, group_body, 0)

    return k


def kernel(q, v, embeddings):
    sq, sv = q.shape[1], v.shape[1]
    K, D = embeddings.shape
    k = _build_sc_kernel(sq, sv, K, D)
    return k(embeddings)


# speedup vs baseline: 1.3326x; 1.3326x over previous
import functools

import jax
import jax.numpy as jnp
from jax import lax
from jax.experimental import pallas as pl
from jax.experimental.pallas import tpu as pltpu
from jax.experimental.pallas import tpu_sc as plsc

CH = 2048
GROUP = 8


def _h_pattern_kernel(emb_ref, h_ref, *, K, D, W, X):
    nxb = X // D
    m = jax.lax.broadcasted_iota(jnp.int32, (8 * nxb, K), 0)
    kk = jax.lax.broadcasted_iota(jnp.int32, (8 * nxb, K), 1)
    r = m // nxb
    xb = m - r * nxb
    pos = jnp.clip(xb - r - W // D, 0, K - 1)
    oh = (kk == pos).astype(jnp.float32)
    res = jax.lax.dot_general(oh, emb_ref[...], (((1,), (0,)), ((), ())),
                              preferred_element_type=jnp.float32)
    h_ref[...] = res.reshape(8, nxb, D)


def _build_sc_kernel(sq, sv, D, X, off0):
    row_words = sv * D
    n_groups8 = sq // 8
    info = plsc.get_sparse_core_info()
    NC = info.num_cores
    NW = NC * info.num_subcores
    g_per_w = n_groups8 // NW
    chunks = row_words // CH
    n_rounds = g_per_w * chunks // GROUP

    mesh = plsc.VectorSubcoreMesh(core_axis_name="c", subcore_axis_name="s")

    @functools.partial(
        pl.kernel, mesh=mesh,
        out_type=jax.ShapeDtypeStruct((sq, row_words), jnp.float32),
        scratch_types=[
            pltpu.VMEM((8, X), jnp.float32),
            pltpu.SemaphoreType.DMA,
        ],
    )
    def k(h_hbm, out_hbm, h_ref, sem):
        wid = lax.axis_index("s") * NC + lax.axis_index("c")
        pltpu.sync_copy(h_hbm, h_ref)
        base_g = wid * g_per_w

        def round_body(it, _):
            copies = []
            for b in range(GROUP):
                t = it * GROUP + b
                g = t // chunks
                c = t % chunks
                a = base_g + g
                c0 = pl.multiple_of(c * CH, 128)
                xoff = pl.multiple_of(
                    jnp.clip(c0 - 8 * a * D + off0, 0, X - CH), 128)
                row0 = pl.multiple_of(8 * a, 8)
                copies.append(pltpu.async_copy(
                    h_ref.at[pl.ds(0, 8), pl.ds(xoff, CH)],
                    out_hbm.at[pl.ds(row0, 8), pl.ds(c0, CH)],
                    sem))
            for cp in copies:
                cp.wait()
            return 0

        lax.fori_loop(0, n_rounds, round_body, 0)

    return k


def kernel(q, v, embeddings):
    sq, sv = q.shape[1], v.shape[1]
    K, D = embeddings.shape
    mp = (K - 1) // 2
    W = CH
    X = W + K * D + CH + 7 * D
    X = -(-X // 128) * 128

    h3 = pl.pallas_call(
        functools.partial(_h_pattern_kernel, K=K, D=D, W=W, X=X),
        out_shape=jax.ShapeDtypeStruct((8, X // D, D), jnp.float32),
    )(embeddings)
    h2d = h3.reshape(8, X)

    off0 = (W // D + mp) * D
    sck = _build_sc_kernel(sq, sv, D, X, off0)
    out2d = sck(h2d)
    return out2d.reshape(sq, sv, D)


# speedup vs baseline: 5.4439x; 4.0851x over previous
import functools

import jax
import jax.numpy as jnp
from jax import lax
from jax.experimental import pallas as pl
from jax.experimental.pallas import tpu as pltpu
from jax.experimental.pallas import tpu_sc as plsc

MB = 13


def _band_blocks_kernel(emb_ref, out_ref, *, K, D, NT):
    step = pl.program_id(0)
    for mm in range(MB):
        m = step * MB + mm
        kk = jax.lax.broadcasted_iota(jnp.int32, (K, 256), 0)
        x = jax.lax.broadcasted_iota(jnp.int32, (K, 256), 1)
        pos = jnp.clip(x - m, 0, K - 1)
        pos = jnp.where(m == NT - 2, 0, pos)
        pos = jnp.where(m == NT - 1, K - 1, pos)
        oh = (kk == pos).astype(jnp.float32)
        res = jax.lax.dot_general(
            emb_ref[...], oh, (((0,), (0,)), ((), ())),
            preferred_element_type=jnp.float32)
        out_ref[pl.ds(D * mm, D), :] = res


def _build_sc_kernel(sq, sv, D, n_shift):
    info = plsc.get_sparse_core_info()
    NC = info.num_cores
    NW = NC * info.num_subcores
    planes_per_w = sq // NW
    ntiles = sv // 128

    mesh = plsc.VectorSubcoreMesh(core_axis_name="c", subcore_axis_name="s")

    @functools.partial(
        pl.kernel, mesh=mesh,
        out_type=jax.ShapeDtypeStruct((sq * D, sv), jnp.float32),
        scratch_types=[
            pltpu.VMEM((D, 512), jnp.float32),
            pltpu.SemaphoreType.DMA,
        ],
    )
    def k(b_hbm, out_hbm, s_ref, sem):
        wid = lax.axis_index("s") * NC + lax.axis_index("c")
        base = wid * planes_per_w

        pltpu.sync_copy(b_hbm.at[pl.ds(D * n_shift, D), pl.ds(0, 128)],
                        s_ref.at[:, pl.ds(0, 128)])
        pltpu.sync_copy(b_hbm.at[pl.ds(D * (n_shift + 1), D), pl.ds(0, 128)],
                        s_ref.at[:, pl.ds(384, 128)])

        def plane_body(p, _):
            i = base + p
            m = lax.rem(i + 64, 128)
            t0 = lax.div(i + 64, 128) - 1
            pltpu.sync_copy(b_hbm.at[pl.ds(pl.multiple_of(D * m, 8), D), :],
                            s_ref.at[:, pl.ds(128, 256)])
            row0 = pl.multiple_of(D * i, 8)
            copies = []
            for t in range(ntiles):
                soff = pl.multiple_of(
                    128 * jnp.clip(t - t0 + 1, 0, 3), 128)
                copies.append(pltpu.async_copy(
                    s_ref.at[:, pl.ds(soff, 128)],
                    out_hbm.at[pl.ds(row0, D), pl.ds(128 * t, 128)],
                    sem))
            for cp in copies:
                cp.wait()
            return 0

        lax.fori_loop(0, planes_per_w, plane_body, 0)

    return k


def kernel(q, v, embeddings):
    sq, sv = q.shape[1], v.shape[1]
    K, D = embeddings.shape
    n_shift = 128
    nt = n_shift + 2

    b_all = pl.pallas_call(
        functools.partial(_band_blocks_kernel, K=K, D=D, NT=nt),
        grid=(nt // MB,),
        in_specs=[pl.BlockSpec((K, D), lambda s: (0, 0))],
        out_specs=pl.BlockSpec((MB * D, 256), lambda s: (s, 0)),
        out_shape=jax.ShapeDtypeStruct((nt * D, 256), jnp.float32),
    )(embeddings)

    sck = _build_sc_kernel(sq, sv, D, n_shift)
    out2 = sck(b_all)
    return out2.reshape(sq, D, sv).transpose(0, 2, 1)
